# baseline (device time: 39828 ns/iter reference)
import jax
import jax.numpy as jnp
from jax import lax
from jax.experimental import pallas as pl
from jax.experimental.pallas import tpu as pltpu

N_DEV = 8
M_PER = 512
K = 4096
N_TOT = 2048
N_PER = 256
NBLK = 8
AROW = 8


def kernel(x, w_mat):
    def body(x_ref, w_ref, out_ref, y_ref, q_ref, recv_ref, amax_ref,
             sc_send, sc_recv, ch_send, ch_recv):
        j = pl.program_id(0)
        my_i = lax.axis_index("i")

        @pl.when(j == 0)
        def _barrier():
            barrier_sem = pltpu.get_barrier_semaphore()
            for d in range(1, N_DEV):
                t = lax.rem(my_i + d, N_DEV)
                pl.semaphore_signal(
                    barrier_sem, inc=1, device_id=(t,),
                    device_id_type=pl.DeviceIdType.MESH,
                )
            pl.semaphore_wait(barrier_sem, N_DEV - 1)

        yj = jnp.dot(x_ref[...], w_ref[...], preferred_element_type=jnp.float32)
        y_ref[:, pl.ds(j * N_PER, N_PER)] = jnp.maximum(yj, 0.0)

        @pl.when(j == NBLK - 1)
        def _comm():
            a_loc = jnp.max(y_ref[...])
            amax_ref[pl.ds(my_i * AROW, AROW), :] = jnp.full(
                (AROW, 128), a_loc, jnp.float32
            )
            for d in range(1, N_DEV):
                t = lax.rem(my_i + d, N_DEV)
                pltpu.make_async_remote_copy(
                    src_ref=amax_ref.at[pl.ds(my_i * AROW, AROW), :],
                    dst_ref=amax_ref.at[pl.ds(my_i * AROW, AROW), :],
                    send_sem=sc_send.at[d],
                    recv_sem=sc_recv.at[d],
                    device_id=(t,),
                    device_id_type=pl.DeviceIdType.MESH,
                ).start()
            for d in range(1, N_DEV):
                s = lax.rem(my_i - d + N_DEV, N_DEV)
                pltpu.make_async_remote_copy(
                    src_ref=amax_ref.at[pl.ds(s * AROW, AROW), :],
                    dst_ref=amax_ref.at[pl.ds(s * AROW, AROW), :],
                    send_sem=sc_send.at[d],
                    recv_sem=sc_recv.at[d],
                    device_id=(s,),
                    device_id_type=pl.DeviceIdType.MESH,
                ).wait_recv()

            amax_g = jnp.max(amax_ref[...])
            scale = amax_g / 127.0

            q = jnp.round(y_ref[...] / scale)
            q = jnp.clip(q, -127.0, 127.0)
            q_ref[...] = q.astype(jnp.int8)

            recv_ref[pl.ds(my_i * M_PER, M_PER), :] = q_ref[
                :, pl.ds(my_i * N_PER, N_PER)
            ]

            for d in range(1, N_DEV):
                t = lax.rem(my_i + d, N_DEV)
                pltpu.make_async_remote_copy(
                    src_ref=q_ref.at[:, pl.ds(t * N_PER, N_PER)],
                    dst_ref=recv_ref.at[pl.ds(my_i * M_PER, M_PER), :],
                    send_sem=ch_send.at[d],
                    recv_sem=ch_recv.at[d],
                    device_id=(t,),
                    device_id_type=pl.DeviceIdType.MESH,
                ).start()
            for d in range(1, N_DEV):
                s = lax.rem(my_i - d + N_DEV, N_DEV)
                pltpu.make_async_remote_copy(
                    src_ref=q_ref.at[:, pl.ds(s * N_PER, N_PER)],
                    dst_ref=recv_ref.at[pl.ds(s * M_PER, M_PER), :],
                    send_sem=ch_send.at[d],
                    recv_sem=ch_recv.at[d],
                    device_id=(s,),
                    device_id_type=pl.DeviceIdType.MESH,
                ).wait_recv()

            for d in range(1, N_DEV):
                t = lax.rem(my_i + d, N_DEV)
                pltpu.make_async_remote_copy(
                    src_ref=q_ref.at[:, pl.ds(t * N_PER, N_PER)],
                    dst_ref=recv_ref.at[pl.ds(my_i * M_PER, M_PER), :],
                    send_sem=ch_send.at[d],
                    recv_sem=ch_recv.at[d],
                    device_id=(t,),
                    device_id_type=pl.DeviceIdType.MESH,
                ).wait_send()
                pltpu.make_async_remote_copy(
                    src_ref=amax_ref.at[pl.ds(my_i * AROW, AROW), :],
                    dst_ref=amax_ref.at[pl.ds(my_i * AROW, AROW), :],
                    send_sem=sc_send.at[d],
                    recv_sem=sc_recv.at[d],
                    device_id=(t,),
                    device_id_type=pl.DeviceIdType.MESH,
                ).wait_send()

            out_ref[...] = recv_ref[...].astype(jnp.float32) * scale

    return pl.pallas_call(
        body,
        grid=(NBLK,),
        in_specs=[
            pl.BlockSpec((M_PER, K), lambda j: (0, 0)),
            pl.BlockSpec((K, N_PER), lambda j: (0, j)),
        ],
        out_specs=pl.BlockSpec((N_DEV * M_PER, N_PER), lambda j: (0, 0)),
        out_shape=jax.ShapeDtypeStruct((N_DEV * M_PER, N_PER), jnp.float32),
        scratch_shapes=[
            pltpu.VMEM((M_PER, N_TOT), jnp.float32),
            pltpu.VMEM((M_PER, N_TOT), jnp.int8),
            pltpu.VMEM((N_DEV * M_PER, N_PER), jnp.int8),
            pltpu.VMEM((N_DEV * AROW, 128), jnp.float32),
            pltpu.SemaphoreType.DMA((N_DEV,)),
            pltpu.SemaphoreType.DMA((N_DEV,)),
            pltpu.SemaphoreType.DMA((N_DEV,)),
            pltpu.SemaphoreType.DMA((N_DEV,)),
        ],
        compiler_params=pltpu.CompilerParams(
            dimension_semantics=("arbitrary",),
            collective_id=0,
        ),
    )(x, w_mat)


# device time: 24973 ns/iter; 1.5948x vs baseline; 1.5948x over previous
import os

import jax
import jax.numpy as jnp
from jax import lax
from jax.experimental import pallas as pl
from jax.experimental.pallas import tpu as pltpu

_VARIANT = os.environ.get("KERNEL_VARIANT", "full")

N_DEV = 8
M_PER = 512
K = 4096
N_TOT = 2048
N_PER = 256
NBLK = 8
AROW = 8


def kernel(x, w_mat):
    def body(x_ref, w_ref, out_ref, y_ref, q_ref, recv_ref, amax_ref,
             sc_send, sc_recv, ch_send, ch_recv):
        j = pl.program_id(0)
        my_i = lax.axis_index("i")

        if _VARIANT != "nocomm":
            @pl.when(j == 0)
            def _barrier():
                barrier_sem = pltpu.get_barrier_semaphore()
                for d in range(1, N_DEV):
                    t = lax.rem(my_i + d, N_DEV)
                    pl.semaphore_signal(
                        barrier_sem, inc=1, device_id=(t,),
                        device_id_type=pl.DeviceIdType.MESH,
                    )
                pl.semaphore_wait(barrier_sem, N_DEV - 1)

        yj = jnp.dot(x_ref[...], w_ref[...], preferred_element_type=jnp.float32)
        y_ref[:, pl.ds(j * N_PER, N_PER)] = jnp.maximum(yj, 0.0)

        @pl.when(j == NBLK - 1)
        def _comm():
            a_loc = jnp.max(y_ref[...])
            if _VARIANT == "full":
                amax_ref[pl.ds(my_i * AROW, AROW), :] = jnp.full(
                    (AROW, 128), a_loc, jnp.float32
                )
                for d in range(1, N_DEV):
                    t = lax.rem(my_i + d, N_DEV)
                    pltpu.make_async_remote_copy(
                        src_ref=amax_ref.at[pl.ds(my_i * AROW, AROW), :],
                        dst_ref=amax_ref.at[pl.ds(my_i * AROW, AROW), :],
                        send_sem=sc_send.at[d],
                        recv_sem=sc_recv.at[d],
                        device_id=(t,),
                        device_id_type=pl.DeviceIdType.MESH,
                    ).start()
                for d in range(1, N_DEV):
                    s = lax.rem(my_i - d + N_DEV, N_DEV)
                    pltpu.make_async_remote_copy(
                        src_ref=amax_ref.at[pl.ds(s * AROW, AROW), :],
                        dst_ref=amax_ref.at[pl.ds(s * AROW, AROW), :],
                        send_sem=sc_send.at[d],
                        recv_sem=sc_recv.at[d],
                        device_id=(s,),
                        device_id_type=pl.DeviceIdType.MESH,
                    ).wait_recv()
                amax_g = jnp.max(amax_ref[...])
            else:
                amax_g = a_loc
            scale = amax_g / 127.0

            q = jnp.round(y_ref[...] / scale)
            q = jnp.clip(q, -127.0, 127.0)
            q_ref[...] = q.astype(jnp.int8)

            recv_ref[pl.ds(my_i * M_PER, M_PER), :] = q_ref[
                :, pl.ds(my_i * N_PER, N_PER)
            ]

            for d in range(1, N_DEV) if _VARIANT != "nocomm" else []:
                t = lax.rem(my_i + d, N_DEV)
                pltpu.make_async_remote_copy(
                    src_ref=q_ref.at[:, pl.ds(t * N_PER, N_PER)],
                    dst_ref=recv_ref.at[pl.ds(my_i * M_PER, M_PER), :],
                    send_sem=ch_send.at[d],
                    recv_sem=ch_recv.at[d],
                    device_id=(t,),
                    device_id_type=pl.DeviceIdType.MESH,
                ).start()
            for d in range(1, N_DEV) if _VARIANT != "nocomm" else []:
                s = lax.rem(my_i - d + N_DEV, N_DEV)
                pltpu.make_async_remote_copy(
                    src_ref=q_ref.at[:, pl.ds(s * N_PER, N_PER)],
                    dst_ref=recv_ref.at[pl.ds(s * M_PER, M_PER), :],
                    send_sem=ch_send.at[d],
                    recv_sem=ch_recv.at[d],
                    device_id=(s,),
                    device_id_type=pl.DeviceIdType.MESH,
                ).wait_recv()

            for d in range(1, N_DEV) if _VARIANT != "nocomm" else []:
                t = lax.rem(my_i + d, N_DEV)
                pltpu.make_async_remote_copy(
                    src_ref=q_ref.at[:, pl.ds(t * N_PER, N_PER)],
                    dst_ref=recv_ref.at[pl.ds(my_i * M_PER, M_PER), :],
                    send_sem=ch_send.at[d],
                    recv_sem=ch_recv.at[d],
                    device_id=(t,),
                    device_id_type=pl.DeviceIdType.MESH,
                ).wait_send()
                if _VARIANT == "full":
                    pltpu.make_async_remote_copy(
                        src_ref=amax_ref.at[pl.ds(my_i * AROW, AROW), :],
                        dst_ref=amax_ref.at[pl.ds(my_i * AROW, AROW), :],
                        send_sem=sc_send.at[d],
                        recv_sem=sc_recv.at[d],
                        device_id=(t,),
                        device_id_type=pl.DeviceIdType.MESH,
                    ).wait_send()

            out_ref[...] = recv_ref[...].astype(jnp.float32) * scale

    return pl.pallas_call(
        body,
        grid=(NBLK,),
        in_specs=[
            pl.BlockSpec((M_PER, K), lambda j: (0, 0)),
            pl.BlockSpec((K, N_PER), lambda j: (0, j)),
        ],
        out_specs=pl.BlockSpec((N_DEV * M_PER, N_PER), lambda j: (0, 0)),
        out_shape=jax.ShapeDtypeStruct((N_DEV * M_PER, N_PER), jnp.float32),
        scratch_shapes=[
            pltpu.VMEM((M_PER, N_TOT), jnp.float32),
            pltpu.VMEM((M_PER, N_TOT), jnp.int8),
            pltpu.VMEM((N_DEV * M_PER, N_PER), jnp.int8),
            pltpu.VMEM((N_DEV * AROW, 128), jnp.float32),
            pltpu.SemaphoreType.DMA((N_DEV,)),
            pltpu.SemaphoreType.DMA((N_DEV,)),
            pltpu.SemaphoreType.DMA((N_DEV,)),
            pltpu.SemaphoreType.DMA((N_DEV,)),
        ],
        compiler_params=pltpu.CompilerParams(
            dimension_semantics=("arbitrary",),
            collective_id=None if _VARIANT == "nocomm" else 0,
        ),
    )(x, w_mat)
